# baseline (speedup 1.0000x reference)
import functools

import jax
import jax.numpy as jnp
from jax import lax
from jax.experimental import pallas as pl
from jax.experimental.pallas import tpu as pltpu
from jax.experimental.pallas import tpu_sc as plsc

_NC = 2
_NS = 16
_NW = _NC * _NS


def _sc_gather(user_ids, item_ids, user_weight, user_bias, item_weight, item_bias):
    B = user_ids.shape[0]
    D = user_weight.shape[1]
    b_per_w = B // _NW

    mesh = plsc.VectorSubcoreMesh(core_axis_name="c", subcore_axis_name="s")

    @functools.partial(
        pl.kernel,
        mesh=mesh,
        out_type=[
            jax.ShapeDtypeStruct((B, D), jnp.float32),
            jax.ShapeDtypeStruct((B, 1), jnp.float32),
            jax.ShapeDtypeStruct((B, D), jnp.float32),
            jax.ShapeDtypeStruct((B, 1), jnp.float32),
        ],
        scratch_types=[
            pltpu.VMEM((b_per_w,), jnp.int32),
            pltpu.VMEM((b_per_w,), jnp.int32),
            pltpu.VMEM((b_per_w, D), jnp.float32),
            pltpu.VMEM((b_per_w, 1), jnp.float32),
            pltpu.VMEM((b_per_w, D), jnp.float32),
            pltpu.VMEM((b_per_w, 1), jnp.float32),
            pltpu.SemaphoreType.DMA,
        ],
    )
    def gather(uids_hbm, iids_hbm, uw_hbm, ub_hbm, iw_hbm, ib_hbm,
               uw_out, ub_out, iw_out, ib_out,
               uidx_v, iidx_v, uw_v, ub_v, iw_v, ib_v, sem):
        wid = lax.axis_index("s") * _NC + lax.axis_index("c")
        base = wid * b_per_w
        pltpu.sync_copy(uids_hbm.at[pl.ds(base, b_per_w)], uidx_v)
        pltpu.sync_copy(iids_hbm.at[pl.ds(base, b_per_w)], iidx_v)

        def group(g, carry):
            uvec = uidx_v[pl.ds(g * 16, 16)]
            ivec = iidx_v[pl.ds(g * 16, 16)]
            for j in range(16):
                i = g * 16 + j
                r = uvec[j]
                s = ivec[j]
                pltpu.async_copy(uw_hbm.at[pl.ds(r, 1)], uw_v.at[pl.ds(i, 1)], sem)
                pltpu.async_copy(iw_hbm.at[pl.ds(s, 1)], iw_v.at[pl.ds(i, 1)], sem)
                pltpu.async_copy(ub_hbm.at[pl.ds(r, 1)], ub_v.at[pl.ds(i, 1)], sem)
                pltpu.async_copy(ib_hbm.at[pl.ds(s, 1)], ib_v.at[pl.ds(i, 1)], sem)
            return carry

        lax.fori_loop(0, b_per_w // 16, group, 0)
        pltpu.make_async_copy(uw_hbm.at[pl.ds(0, b_per_w)], uw_v, sem).wait()
        pltpu.make_async_copy(iw_hbm.at[pl.ds(0, b_per_w)], iw_v, sem).wait()
        pltpu.make_async_copy(ub_hbm.at[pl.ds(0, b_per_w)], ub_v, sem).wait()
        pltpu.make_async_copy(ib_hbm.at[pl.ds(0, b_per_w)], ib_v, sem).wait()

        pltpu.sync_copy(uw_v, uw_out.at[pl.ds(base, b_per_w)])
        pltpu.sync_copy(ub_v, ub_out.at[pl.ds(base, b_per_w)])
        pltpu.sync_copy(iw_v, iw_out.at[pl.ds(base, b_per_w)])
        pltpu.sync_copy(ib_v, ib_out.at[pl.ds(base, b_per_w)])

    return gather(user_ids, item_ids, user_weight, user_bias,
                  item_weight, item_bias)


def _tc_rating(uw, iw, ub, ib_row, bias11):
    B, D = uw.shape
    BM = 512

    def body(uw_ref, iw_ref, ub_ref, ib_ref, b_ref, out_ref):
        acc = lax.dot_general(
            uw_ref[...], iw_ref[...], (((1,), (1,)), ((), ())),
            preferred_element_type=jnp.float32)
        i = pl.program_id(0)
        ub_blk = ub_ref[pl.ds(i * BM, BM), :]
        out_ref[...] = acc + ub_blk + ib_ref[...] + b_ref[0, 0]

    return pl.pallas_call(
        body,
        grid=(B // BM,),
        in_specs=[
            pl.BlockSpec((BM, D), lambda i: (i, 0)),
            pl.BlockSpec((B, D), lambda i: (0, 0)),
            pl.BlockSpec((B, 1), lambda i: (0, 0)),
            pl.BlockSpec((1, B), lambda i: (0, 0)),
            pl.BlockSpec((1, 1), lambda i: (0, 0)),
        ],
        out_specs=pl.BlockSpec((BM, B), lambda i: (i, 0)),
        out_shape=jax.ShapeDtypeStruct((B, B), jnp.float32),
    )(uw, iw, ub, ib_row, bias11)


def kernel(user_ids, item_ids, user_weight, user_bias, item_weight, item_bias, bias):
    uw, ub, iw, ib = _sc_gather(user_ids, item_ids, user_weight, user_bias,
                                item_weight, item_bias)
    return uw @ iw.T + ub + ib.T + bias


# speedup vs baseline: 1.1388x; 1.1388x over previous
import functools

import jax
import jax.numpy as jnp
from jax import lax
from jax.experimental import pallas as pl
from jax.experimental.pallas import tpu as pltpu
from jax.experimental.pallas import tpu_sc as plsc

_NC = 2
_NS = 16
_NW = _NC * _NS


def _sc_gather(user_ids, item_ids, user_weight, ub1, item_weight, ib1):
    B = user_ids.shape[0]
    D = user_weight.shape[1]
    b_per_w = B // _NW

    mesh = plsc.VectorSubcoreMesh(core_axis_name="c", subcore_axis_name="s")

    @functools.partial(
        pl.kernel,
        mesh=mesh,
        out_type=[
            jax.ShapeDtypeStruct((B, D), jnp.float32),
            jax.ShapeDtypeStruct((B,), jnp.float32),
            jax.ShapeDtypeStruct((B, D), jnp.float32),
            jax.ShapeDtypeStruct((B,), jnp.float32),
        ],
        scratch_types=[
            pltpu.VMEM((b_per_w,), jnp.int32),
            pltpu.VMEM((b_per_w,), jnp.int32),
            pltpu.VMEM((b_per_w, D), jnp.float32),
            pltpu.VMEM((b_per_w, D), jnp.float32),
            pltpu.VMEM((b_per_w,), jnp.float32),
            pltpu.VMEM((b_per_w,), jnp.float32),
            pltpu.SemaphoreType.DMA,
        ],
        compiler_params=pltpu.CompilerParams(use_tc_tiling_on_sc=False),
    )
    def gather(uids_hbm, iids_hbm, uw_hbm, ub1_hbm, iw_hbm, ib1_hbm,
               uw_out, ub_out, iw_out, ib_out,
               uidx_v, iidx_v, uw_v, iw_v, ubg, ibg, sem):
        wid = lax.axis_index("s") * _NC + lax.axis_index("c")
        base = wid * b_per_w
        pltpu.sync_copy(uids_hbm.at[pl.ds(base, b_per_w)], uidx_v)
        pltpu.sync_copy(iids_hbm.at[pl.ds(base, b_per_w)], iidx_v)
        cu = pltpu.async_copy(uw_hbm.at[uidx_v], uw_v, sem)
        ci = pltpu.async_copy(iw_hbm.at[iidx_v], iw_v, sem)
        cub = pltpu.async_copy(ub1_hbm.at[uidx_v], ubg, sem)
        cib = pltpu.async_copy(ib1_hbm.at[iidx_v], ibg, sem)
        cu.wait()
        ci.wait()
        cub.wait()
        cib.wait()
        pltpu.sync_copy(uw_v, uw_out.at[pl.ds(base, b_per_w)])
        pltpu.sync_copy(iw_v, iw_out.at[pl.ds(base, b_per_w)])
        pltpu.sync_copy(ubg, ub_out.at[pl.ds(base, b_per_w)])
        pltpu.sync_copy(ibg, ib_out.at[pl.ds(base, b_per_w)])

    return gather(user_ids, item_ids, user_weight, ub1, item_weight, ib1)


def _tc_rating(uw_g, iw_g, ub_g, ib_g, bias11):
    B, D = uw_g.shape
    BM = 512

    def body(uw_ref, iw_ref, ub_ref, ib_ref, b_ref, out_ref):
        acc = lax.dot_general(
            uw_ref[...], iw_ref[...], (((1,), (1,)), ((), ())),
            preferred_element_type=jnp.float32)
        i = pl.program_id(0)
        ub_blk = ub_ref[pl.ds(i * BM, BM), :]
        out_ref[...] = acc + ub_blk + ib_ref[...] + b_ref[0, 0]

    return pl.pallas_call(
        body,
        grid=(B // BM,),
        in_specs=[
            pl.BlockSpec((BM, D), lambda i: (i, 0)),
            pl.BlockSpec((B, D), lambda i: (0, 0)),
            pl.BlockSpec((B, 1), lambda i: (0, 0)),
            pl.BlockSpec((1, B), lambda i: (0, 0)),
            pl.BlockSpec((1, 1), lambda i: (0, 0)),
        ],
        out_specs=pl.BlockSpec((BM, B), lambda i: (i, 0)),
        out_shape=jax.ShapeDtypeStruct((B, B), jnp.float32),
    )(uw_g, iw_g, ub_g, ib_g, bias11)


def kernel(user_ids, item_ids, user_weight, user_bias, item_weight, item_bias, bias):
    B = user_ids.shape[0]
    ub1 = jnp.reshape(user_bias, (-1,))
    ib1 = jnp.reshape(item_bias, (-1,))
    uw_g, ub_g, iw_g, ib_g = _sc_gather(user_ids, item_ids, user_weight, ub1,
                                        item_weight, ib1)
    ub2 = jnp.reshape(ub_g, (B, 1))
    ib2 = jnp.reshape(ib_g, (1, B))
    bias11 = jnp.reshape(bias, (1, 1))
    return _tc_rating(uw_g, iw_g, ub2, ib2, bias11)
